# Initial kernel scaffold; baseline (speedup 1.0000x reference)
import jax
import jax.numpy as jnp
from jax.experimental import pallas as pl


def kernel(X, ind, W):
    raise NotImplementedError("write your pallas kernel here")


# speedup vs baseline: 3.7367x; 3.7367x over previous
import functools

import jax
import jax.numpy as jnp
from jax import lax
from jax.experimental import pallas as pl
from jax.experimental.pallas import tpu as pltpu
from jax.experimental.pallas import tpu_sc as plsc

_NUM_CORES = 2
_NUM_SUBCORES = 16
_NUM_WORKERS = _NUM_CORES * _NUM_SUBCORES
_CHUNK = 32


def _sc_gather(x_flat, idx):
    V, D = x_flat.shape
    _, n_chunks, _ = idx.shape
    rows_per_w = n_chunks * _CHUNK
    total_rows = _NUM_WORKERS * rows_per_w

    mesh = plsc.VectorSubcoreMesh(
        core_axis_name="c",
        subcore_axis_name="s",
        num_cores=_NUM_CORES,
        num_subcores=_NUM_SUBCORES,
    )

    @functools.partial(
        pl.kernel,
        mesh=mesh,
        out_type=jax.ShapeDtypeStruct((total_rows, D), jnp.float32),
        scratch_types=[
            pltpu.VMEM((n_chunks, _CHUNK), jnp.int32),
            pltpu.VMEM((_CHUNK, D), jnp.float32),
            pltpu.SemaphoreType.DMA,
        ],
    )
    def gather_kernel(idx_hbm, x_hbm, out_hbm, idx_v, rows_v, gsem):
        wid = lax.axis_index("s") * _NUM_CORES + lax.axis_index("c")
        base = wid * rows_per_w
        pltpu.sync_copy(idx_hbm.at[wid], idx_v)

        def body(c, _):
            pltpu.async_copy(x_hbm.at[idx_v.at[c]], rows_v, gsem).wait()
            pltpu.sync_copy(rows_v, out_hbm.at[pl.ds(base + c * _CHUNK, _CHUNK)])
            return _

        lax.fori_loop(0, n_chunks, body, None)

    return gather_kernel(idx, x_flat)


def _tc_matmul(xg, w):
    B, H, K, D = xg.shape
    HD = w.shape[2]

    def body(xg_ref, w_ref, out_ref):
        out_ref[0, 0] = lax.dot_general(
            xg_ref[0, 0],
            w_ref[0],
            (((1,), (0,)), ((), ())),
            preferred_element_type=jnp.float32,
        )

    return pl.pallas_call(
        body,
        grid=(H, B),
        in_specs=[
            pl.BlockSpec((1, 1, K, D), lambda h, b: (b, h, 0, 0)),
            pl.BlockSpec((1, D, HD), lambda h, b: (h, 0, 0)),
        ],
        out_specs=pl.BlockSpec((1, 1, K, HD), lambda h, b: (b, h, 0, 0)),
        out_shape=jax.ShapeDtypeStruct((B, H, K, HD), jnp.float32),
    )(xg, w)


def kernel(X, ind, W):
    B, N, D = X.shape
    _, H, K = ind.shape
    HD = W.shape[2]

    total_rows = B * H * K
    rows_per_w = total_rows // _NUM_WORKERS
    n_chunks = rows_per_w // _CHUNK

    idx = (
        ind.astype(jnp.int32) + (jnp.arange(B, dtype=jnp.int32) * N)[:, None, None]
    ).reshape(_NUM_WORKERS, n_chunks, _CHUNK)

    xg = _sc_gather(X.reshape(B * N, D), idx)
    return _tc_matmul(xg.reshape(B, H, K, D), W)
